# baseline (speedup 1.0000x reference)
import jax
import jax.numpy as jnp
from jax import lax
from jax.experimental import pallas as pl
from jax.experimental.pallas import tpu as pltpu
from jax.experimental.pallas import tpu_sc as plsc

INFEATURES = 256
CAPACITY = 8192
NC, NS, L = 2, 16, 16
NW = NC * NS
R = CAPACITY // NW
NG = R // L
NCHUNK = 4
CROWS = R // NCHUNK
PSTRIDE = L + 1
EPS = 1e-8


def _rsqrt(n):
    i = lax.bitcast_convert_type(n, jnp.int32)
    y = lax.bitcast_convert_type(jnp.int32(0x5F3759DF) - (i >> 1), jnp.float32)
    for _ in range(3):
        y = y * (jnp.float32(1.5) - jnp.float32(0.5) * n * y * y)
    return y


def _sc_body(x_hbm, mem_hbm, val_out, idx_out, x_v, buf, pa, pn, vb, ib,
             s0, s1, s2, s3):
    wid = lax.axis_index("s") * NC + lax.axis_index("c")
    base = wid * R

    pltpu.sync_copy(x_hbm, x_v)
    copies = [
        pltpu.async_copy(
            mem_hbm.at[pl.ds(base + k * CROWS, CROWS)],
            buf.at[pl.ds(k * CROWS, CROWS)],
            sem,
        )
        for k, sem in enumerate((s0, s1, s2, s3))
    ]

    lane = lax.iota(jnp.int32, L)
    colbase = lane * PSTRIDE
    xvs = [x_v[pl.ds(L * j, L)] for j in range(INFEATURES // L)]

    def group_body(g, carry):
        bv, bi = carry
        for k in range(NCHUNK):
            @pl.when(g == k * (NG // NCHUNK))
            def _(k=k):
                copies[k].wait()
        for r16 in range(L):
            r = g * L + r16
            a0 = jnp.zeros((L,), jnp.float32)
            a1 = jnp.zeros((L,), jnp.float32)
            n0 = jnp.zeros((L,), jnp.float32)
            n1 = jnp.zeros((L,), jnp.float32)
            for j in range(INFEATURES // L):
                v = buf[r, pl.ds(L * j, L)]
                if j % 2 == 0:
                    a0 = a0 + v * xvs[j]
                    n0 = n0 + v * v
                else:
                    a1 = a1 + v * xvs[j]
                    n1 = n1 + v * v
            pa[pl.ds(r16 * PSTRIDE, L)] = a0 + a1
            pn[pl.ds(r16 * PSTRIDE, L)] = n0 + n1
        dotv = plsc.load_gather(pa, [colbase])
        nrmv = plsc.load_gather(pn, [colbase])
        for c in range(1, L):
            dotv = dotv + plsc.load_gather(pa, [colbase + c])
            nrmv = nrmv + plsc.load_gather(pn, [colbase + c])
        mn = jnp.maximum(nrmv * _rsqrt(nrmv), EPS)
        d = dotv / mn
        upd = d > bv
        bi = jnp.where(upd, lane + g * L, bi)
        bv = jnp.where(upd, d, bv)
        return bv, bi

    bv0 = jnp.full((L,), -jnp.inf, jnp.float32)
    bi0 = jnp.zeros((L,), jnp.int32)
    bv, bi = lax.fori_loop(0, NG, group_body, (bv0, bi0))
    vb[...] = bv
    ib[...] = bi
    pltpu.sync_copy(vb, val_out.at[wid])
    pltpu.sync_copy(ib, idx_out.at[wid])


def _merge_body(x_ref, val_ref, idx_ref, out_ref):
    vals = val_ref[...]
    idxs = idx_ref[...] + lax.broadcasted_iota(jnp.int32, (NW, L), 0) * R
    m = jnp.max(vals)
    big = jnp.int32(jnp.iinfo(jnp.int32).max)
    idx = jnp.min(jnp.where(vals == m, idxs, big))
    xv = x_ref[...]
    xn = jnp.maximum(jnp.sqrt(jnp.sum(xv * xv)), jnp.float32(EPS))
    rows = lax.broadcasted_iota(jnp.int32, (64, 128), 0)
    cols = lax.broadcasted_iota(jnp.int32, (64, 128), 1)
    lin = rows * 128 + cols
    out_ref[...] = jnp.where(lin == idx, m / xn, jnp.float32(0.0))


@jax.jit
def kernel(x, memory):
    mesh = plsc.VectorSubcoreMesh(core_axis_name="c", subcore_axis_name="s")
    sc = pl.kernel(
        _sc_body,
        out_type=(
            jax.ShapeDtypeStruct((NW, L), jnp.float32),
            jax.ShapeDtypeStruct((NW, L), jnp.int32),
        ),
        mesh=mesh,
        compiler_params=pltpu.CompilerParams(needs_layout_passes=False),
        scratch_types=[
            pltpu.VMEM((INFEATURES,), jnp.float32),
            pltpu.VMEM((R, INFEATURES), jnp.float32),
            pltpu.VMEM((L * PSTRIDE,), jnp.float32),
            pltpu.VMEM((L * PSTRIDE,), jnp.float32),
            pltpu.VMEM((L,), jnp.float32),
            pltpu.VMEM((L,), jnp.int32),
            pltpu.SemaphoreType.DMA,
            pltpu.SemaphoreType.DMA,
            pltpu.SemaphoreType.DMA,
            pltpu.SemaphoreType.DMA,
        ],
    )
    cand_val, cand_idx = sc(x, memory)
    out2d = pl.pallas_call(
        _merge_body,
        out_shape=jax.ShapeDtypeStruct((64, 128), jnp.float32),
    )(x.reshape(2, 128), cand_val, cand_idx)
    return out2d.reshape(CAPACITY)


# speedup vs baseline: 1.0716x; 1.0716x over previous
import jax
import jax.numpy as jnp
from jax import lax
from jax.experimental import pallas as pl
from jax.experimental.pallas import tpu as pltpu
from jax.experimental.pallas import tpu_sc as plsc

INFEATURES = 256
CAPACITY = 8192
NC, NS, L = 2, 16, 16
NW = NC * NS
R = CAPACITY // NW
NG = R // L
NCHUNK = 4
CROWS = R // NCHUNK
PSTRIDE = L + 1
EPS = 1e-8


def _rsqrt(n):
    i = lax.bitcast_convert_type(n, jnp.int32)
    y = lax.bitcast_convert_type(jnp.int32(0x5F3759DF) - (i >> 1), jnp.float32)
    for _ in range(3):
        y = y * (jnp.float32(1.5) - jnp.float32(0.5) * n * y * y)
    return y


def _sc_body(x_hbm, mem_hbm, val_out, idx_out, x_v, buf, pa, pn, vb, ib,
             s0, s1, s2, s3):
    wid = lax.axis_index("s") * NC + lax.axis_index("c")
    base = wid * R

    copies = [
        pltpu.async_copy(
            mem_hbm.at[pl.ds(base + k * CROWS, CROWS)],
            buf.at[pl.ds(k * CROWS, CROWS)],
            sem,
        )
        for k, sem in enumerate((s0, s1, s2, s3))
    ]
    pltpu.sync_copy(x_hbm, x_v)

    lane = lax.iota(jnp.int32, L)
    colbase = lane * PSTRIDE
    xvs = [x_v[pl.ds(L * j, L)] for j in range(INFEATURES // L)]

    def group_body(g, carry):
        bv, bi = carry
        for k in range(NCHUNK):
            @pl.when(g == k * (NG // NCHUNK))
            def _(k=k):
                copies[k].wait()
        for r16 in range(0, L, 2):
            rA = g * L + r16
            rB = rA + 1
            aA0 = aA1 = nA0 = nA1 = jnp.zeros((L,), jnp.float32)
            aB0 = aB1 = nB0 = nB1 = jnp.zeros((L,), jnp.float32)
            for j in range(INFEATURES // L):
                vA = buf[rA, pl.ds(L * j, L)]
                vB = buf[rB, pl.ds(L * j, L)]
                if j % 2 == 0:
                    aA0 = aA0 + vA * xvs[j]
                    nA0 = nA0 + vA * vA
                    aB0 = aB0 + vB * xvs[j]
                    nB0 = nB0 + vB * vB
                else:
                    aA1 = aA1 + vA * xvs[j]
                    nA1 = nA1 + vA * vA
                    aB1 = aB1 + vB * xvs[j]
                    nB1 = nB1 + vB * vB
            pa[pl.ds(r16 * PSTRIDE, L)] = aA0 + aA1
            pn[pl.ds(r16 * PSTRIDE, L)] = nA0 + nA1
            pa[pl.ds((r16 + 1) * PSTRIDE, L)] = aB0 + aB1
            pn[pl.ds((r16 + 1) * PSTRIDE, L)] = nB0 + nB1
        dotv = plsc.load_gather(pa, [colbase])
        nrmv = plsc.load_gather(pn, [colbase])
        for c in range(1, L):
            dotv = dotv + plsc.load_gather(pa, [colbase + c])
            nrmv = nrmv + plsc.load_gather(pn, [colbase + c])
        mn = jnp.maximum(nrmv * _rsqrt(nrmv), EPS)
        d = dotv / mn
        upd = d > bv
        bi = jnp.where(upd, lane + g * L, bi)
        bv = jnp.where(upd, d, bv)
        return bv, bi

    bv0 = jnp.full((L,), -jnp.inf, jnp.float32)
    bi0 = jnp.zeros((L,), jnp.int32)
    bv, bi = lax.fori_loop(0, NG, group_body, (bv0, bi0))
    vb[...] = bv
    ib[...] = bi
    pltpu.sync_copy(vb, val_out.at[wid])
    pltpu.sync_copy(ib, idx_out.at[wid])


def _merge_body(x_ref, val_ref, idx_ref, out_ref):
    vals = val_ref[...]
    idxs = idx_ref[...] + lax.broadcasted_iota(jnp.int32, (NW, L), 0) * R
    m = jnp.max(vals)
    big = jnp.int32(jnp.iinfo(jnp.int32).max)
    idx = jnp.min(jnp.where(vals == m, idxs, big))
    xv = x_ref[...]
    xn = jnp.maximum(jnp.sqrt(jnp.sum(xv * xv)), jnp.float32(EPS))
    rows = lax.broadcasted_iota(jnp.int32, (64, 128), 0)
    cols = lax.broadcasted_iota(jnp.int32, (64, 128), 1)
    lin = rows * 128 + cols
    out_ref[...] = jnp.where(lin == idx, m / xn, jnp.float32(0.0))


@jax.jit
def kernel(x, memory):
    mesh = plsc.VectorSubcoreMesh(core_axis_name="c", subcore_axis_name="s")
    sc = pl.kernel(
        _sc_body,
        out_type=(
            jax.ShapeDtypeStruct((NW, L), jnp.float32),
            jax.ShapeDtypeStruct((NW, L), jnp.int32),
        ),
        mesh=mesh,
        compiler_params=pltpu.CompilerParams(needs_layout_passes=False),
        scratch_types=[
            pltpu.VMEM((INFEATURES,), jnp.float32),
            pltpu.VMEM((R, INFEATURES), jnp.float32),
            pltpu.VMEM((L * PSTRIDE,), jnp.float32),
            pltpu.VMEM((L * PSTRIDE,), jnp.float32),
            pltpu.VMEM((L,), jnp.float32),
            pltpu.VMEM((L,), jnp.int32),
            pltpu.SemaphoreType.DMA,
            pltpu.SemaphoreType.DMA,
            pltpu.SemaphoreType.DMA,
            pltpu.SemaphoreType.DMA,
        ],
    )
    cand_val, cand_idx = sc(x, memory)
    out2d = pl.pallas_call(
        _merge_body,
        out_shape=jax.ShapeDtypeStruct((64, 128), jnp.float32),
    )(x.reshape(2, 128), cand_val, cand_idx)
    return out2d.reshape(CAPACITY)


# speedup vs baseline: 1.0944x; 1.0213x over previous
import jax
import jax.numpy as jnp
from jax import lax
from jax.experimental import pallas as pl
from jax.experimental.pallas import tpu as pltpu
from jax.experimental.pallas import tpu_sc as plsc

INFEATURES = 256
CAPACITY = 8192
NC, NS, L = 2, 16, 16
NW = NC * NS
R = CAPACITY // NW
NG = R // L
NCHUNK = 4
CROWS = R // NCHUNK
PSTRIDE = L + 1
EPS = 1e-8


def _rsqrt(n):
    i = lax.bitcast_convert_type(n, jnp.int32)
    y = lax.bitcast_convert_type(jnp.int32(0x5F3759DF) - (i >> 1), jnp.float32)
    for _ in range(3):
        y = y * (jnp.float32(1.5) - jnp.float32(0.5) * n * y * y)
    return y


def _sc_body(x_hbm, mem_hbm, val_out, idx_out, x_v, buf, pa, pn, vb, ib,
             s0, s1, s2, s3):
    wid = lax.axis_index("s") * NC + lax.axis_index("c")
    base = wid * R

    copies = [
        pltpu.async_copy(
            mem_hbm.at[pl.ds(base + k * CROWS, CROWS)],
            buf.at[pl.ds(k * CROWS, CROWS)],
            sem,
        )
        for k, sem in enumerate((s0, s1, s2, s3))
    ]
    pltpu.sync_copy(x_hbm, x_v)

    lane = lax.iota(jnp.int32, L)
    colbase = lane * PSTRIDE
    xvs = [x_v[pl.ds(L * j, L)] for j in range(INFEATURES // L)]

    def group_body(g, carry):
        bv, bi = carry
        for k in range(NCHUNK):
            @pl.when(g == k * (NG // NCHUNK))
            def _(k=k):
                copies[k].wait()
        for r16 in range(0, L, 4):
            rr = [g * L + r16 + i for i in range(4)]
            aa = [jnp.zeros((L,), jnp.float32) for _ in range(4)]
            nn = [jnp.zeros((L,), jnp.float32) for _ in range(4)]
            for j in range(INFEATURES // L):
                for i in range(4):
                    v = buf[rr[i], pl.ds(L * j, L)]
                    aa[i] = aa[i] + v * xvs[j]
                    nn[i] = nn[i] + v * v
            for i in range(4):
                pa[pl.ds((r16 + i) * PSTRIDE, L)] = aa[i]
                pn[pl.ds((r16 + i) * PSTRIDE, L)] = nn[i]
        dotv = plsc.load_gather(pa, [colbase])
        nrmv = plsc.load_gather(pn, [colbase])
        for c in range(1, L):
            dotv = dotv + plsc.load_gather(pa, [colbase + c])
            nrmv = nrmv + plsc.load_gather(pn, [colbase + c])
        mn = jnp.maximum(nrmv * _rsqrt(nrmv), EPS)
        d = dotv / mn
        upd = d > bv
        bi = jnp.where(upd, lane + g * L, bi)
        bv = jnp.where(upd, d, bv)
        return bv, bi

    bv0 = jnp.full((L,), -jnp.inf, jnp.float32)
    bi0 = jnp.zeros((L,), jnp.int32)
    bv, bi = lax.fori_loop(0, NG, group_body, (bv0, bi0))
    vb[...] = bv
    ib[...] = bi
    pltpu.sync_copy(vb, val_out.at[wid])
    pltpu.sync_copy(ib, idx_out.at[wid])


def _merge_body(x_ref, val_ref, idx_ref, out_ref):
    vals = val_ref[...]
    idxs = idx_ref[...] + lax.broadcasted_iota(jnp.int32, (NW, L), 0) * R
    m = jnp.max(vals)
    big = jnp.int32(jnp.iinfo(jnp.int32).max)
    idx = jnp.min(jnp.where(vals == m, idxs, big))
    xv = x_ref[...]
    xn = jnp.maximum(jnp.sqrt(jnp.sum(xv * xv)), jnp.float32(EPS))
    rows = lax.broadcasted_iota(jnp.int32, (64, 128), 0)
    cols = lax.broadcasted_iota(jnp.int32, (64, 128), 1)
    lin = rows * 128 + cols
    out_ref[...] = jnp.where(lin == idx, m / xn, jnp.float32(0.0))


@jax.jit
def kernel(x, memory):
    mesh = plsc.VectorSubcoreMesh(core_axis_name="c", subcore_axis_name="s")
    sc = pl.kernel(
        _sc_body,
        out_type=(
            jax.ShapeDtypeStruct((NW, L), jnp.float32),
            jax.ShapeDtypeStruct((NW, L), jnp.int32),
        ),
        mesh=mesh,
        compiler_params=pltpu.CompilerParams(needs_layout_passes=False),
        scratch_types=[
            pltpu.VMEM((INFEATURES,), jnp.float32),
            pltpu.VMEM((R, INFEATURES), jnp.float32),
            pltpu.VMEM((L * PSTRIDE,), jnp.float32),
            pltpu.VMEM((L * PSTRIDE,), jnp.float32),
            pltpu.VMEM((L,), jnp.float32),
            pltpu.VMEM((L,), jnp.int32),
            pltpu.SemaphoreType.DMA,
            pltpu.SemaphoreType.DMA,
            pltpu.SemaphoreType.DMA,
            pltpu.SemaphoreType.DMA,
        ],
    )
    cand_val, cand_idx = sc(x, memory)
    out2d = pl.pallas_call(
        _merge_body,
        out_shape=jax.ShapeDtypeStruct((64, 128), jnp.float32),
    )(x.reshape(2, 128), cand_val, cand_idx)
    return out2d.reshape(CAPACITY)


# speedup vs baseline: 1.1001x; 1.0052x over previous
import jax
import jax.numpy as jnp
from jax import lax
from jax.experimental import pallas as pl
from jax.experimental.pallas import tpu as pltpu
from jax.experimental.pallas import tpu_sc as plsc

INFEATURES = 256
CAPACITY = 8192
NC, NS, L = 2, 16, 16
NW = NC * NS
R = CAPACITY // NW
NG = R // L
NCHUNK = 4
CROWS = R // NCHUNK
PSTRIDE = L + 1
EPS = 1e-8


def _rsqrt(n):
    i = lax.bitcast_convert_type(n, jnp.int32)
    y = lax.bitcast_convert_type(jnp.int32(0x5F3759DF) - (i >> 1), jnp.float32)
    for _ in range(3):
        y = y * (jnp.float32(1.5) - jnp.float32(0.5) * n * y * y)
    return y


def _sc_body(x_hbm, mem_hbm, val_out, idx_out, x_v, buf, pa, pn, vb, ib,
             s0, s1, s2, s3):
    wid = lax.axis_index("s") * NC + lax.axis_index("c")
    base = wid * R

    copies = [
        pltpu.async_copy(
            mem_hbm.at[pl.ds(base + k * CROWS, CROWS)],
            buf.at[pl.ds(k * CROWS, CROWS)],
            sem,
        )
        for k, sem in enumerate((s0, s1, s2, s3))
    ]
    pltpu.sync_copy(x_hbm, x_v)

    lane = lax.iota(jnp.int32, L)
    colbase = lane * PSTRIDE
    xvs = [x_v[pl.ds(L * j, L)] for j in range(INFEATURES // L)]

    def group_body(g, carry):
        bv, bi = carry
        for k in range(NCHUNK):
            @pl.when(g == k * (NG // NCHUNK))
            def _(k=k):
                copies[k].wait()
        for r16 in range(0, L, 8):
            rr = [g * L + r16 + i for i in range(8)]
            aa = [jnp.zeros((L,), jnp.float32) for _ in range(8)]
            nn = [jnp.zeros((L,), jnp.float32) for _ in range(8)]
            for j in range(INFEATURES // L):
                for i in range(8):
                    v = buf[rr[i], pl.ds(L * j, L)]
                    aa[i] = aa[i] + v * xvs[j]
                    nn[i] = nn[i] + v * v
            for i in range(8):
                pa[pl.ds((r16 + i) * PSTRIDE, L)] = aa[i]
                pn[pl.ds((r16 + i) * PSTRIDE, L)] = nn[i]
        dotv = plsc.load_gather(pa, [colbase])
        nrmv = plsc.load_gather(pn, [colbase])
        for c in range(1, L):
            dotv = dotv + plsc.load_gather(pa, [colbase + c])
            nrmv = nrmv + plsc.load_gather(pn, [colbase + c])
        mn = jnp.maximum(nrmv * _rsqrt(nrmv), EPS)
        d = dotv / mn
        upd = d > bv
        bi = jnp.where(upd, lane + g * L, bi)
        bv = jnp.where(upd, d, bv)
        return bv, bi

    bv0 = jnp.full((L,), -jnp.inf, jnp.float32)
    bi0 = jnp.zeros((L,), jnp.int32)
    bv, bi = lax.fori_loop(0, NG, group_body, (bv0, bi0))
    vb[...] = bv
    ib[...] = bi
    pltpu.sync_copy(vb, val_out.at[wid])
    pltpu.sync_copy(ib, idx_out.at[wid])


def _merge_body(x_ref, val_ref, idx_ref, out_ref):
    vals = val_ref[...]
    idxs = idx_ref[...] + lax.broadcasted_iota(jnp.int32, (NW, L), 0) * R
    m = jnp.max(vals)
    big = jnp.int32(jnp.iinfo(jnp.int32).max)
    idx = jnp.min(jnp.where(vals == m, idxs, big))
    xv = x_ref[...]
    xn = jnp.maximum(jnp.sqrt(jnp.sum(xv * xv)), jnp.float32(EPS))
    rows = lax.broadcasted_iota(jnp.int32, (64, 128), 0)
    cols = lax.broadcasted_iota(jnp.int32, (64, 128), 1)
    lin = rows * 128 + cols
    out_ref[...] = jnp.where(lin == idx, m / xn, jnp.float32(0.0))


@jax.jit
def kernel(x, memory):
    mesh = plsc.VectorSubcoreMesh(core_axis_name="c", subcore_axis_name="s")
    sc = pl.kernel(
        _sc_body,
        out_type=(
            jax.ShapeDtypeStruct((NW, L), jnp.float32),
            jax.ShapeDtypeStruct((NW, L), jnp.int32),
        ),
        mesh=mesh,
        compiler_params=pltpu.CompilerParams(needs_layout_passes=False),
        scratch_types=[
            pltpu.VMEM((INFEATURES,), jnp.float32),
            pltpu.VMEM((R, INFEATURES), jnp.float32),
            pltpu.VMEM((L * PSTRIDE,), jnp.float32),
            pltpu.VMEM((L * PSTRIDE,), jnp.float32),
            pltpu.VMEM((L,), jnp.float32),
            pltpu.VMEM((L,), jnp.int32),
            pltpu.SemaphoreType.DMA,
            pltpu.SemaphoreType.DMA,
            pltpu.SemaphoreType.DMA,
            pltpu.SemaphoreType.DMA,
        ],
    )
    cand_val, cand_idx = sc(x, memory)
    out2d = pl.pallas_call(
        _merge_body,
        out_shape=jax.ShapeDtypeStruct((64, 128), jnp.float32),
    )(x.reshape(2, 128), cand_val, cand_idx)
    return out2d.reshape(CAPACITY)
